# Initial kernel scaffold; baseline (speedup 1.0000x reference)
import functools
import math

import jax
import jax.numpy as jnp
from jax import lax
from jax.experimental import pallas as pl
from jax.experimental.pallas import tpu as pltpu
from jax.experimental.pallas import tpu_sc as plsc

NC = 2
NS = 16
NW = NC * NS
LN = 16

CUTOFF = 5.0
LOG2 = math.log(2.0)


def _ssp(x):
    return jnp.maximum(x, 0.0) + jnp.log1p(jnp.exp(-jnp.abs(x))) - LOG2


def _build_geom(N, E, H, NB, CK):
    EPT = E // NW
    n_ech = EPT // CK
    n_nch = N // CK
    n_rounds = (n_nch + NW - 1) // NW
    mesh = plsc.VectorSubcoreMesh(core_axis_name="c", subcore_axis_name="s")

    @functools.partial(
        pl.kernel,
        out_type=(jax.ShapeDtypeStruct((E,), jnp.float32),
                  jax.ShapeDtypeStruct((N, H), jnp.float32)),
        mesh=mesh,
        scratch_types=[
            pltpu.VMEM((N,), jnp.float32),
            pltpu.VMEM((N,), jnp.float32),
            pltpu.VMEM((N,), jnp.float32),
            pltpu.VMEM((N,), jnp.int32),
            pltpu.VMEM((NB * 9,), jnp.float32),
            pltpu.VMEM((CK,), jnp.int32),
            pltpu.VMEM((CK,), jnp.int32),
            pltpu.VMEM((CK,), jnp.float32),
            pltpu.VMEM((CK,), jnp.float32),
            pltpu.VMEM((CK,), jnp.float32),
            pltpu.VMEM((CK,), jnp.float32),
            pltpu.VMEM((CK,), jnp.int32),
            pltpu.VMEM((CK, H), jnp.float32),
            pltpu.SemaphoreType.DMA,
        ],
    )
    def geom(posT_h, batch_h, cellf_h, src_h, dst_h, shiftsT_h, z_h, emb_h,
             d2_h, h0_h,
             posx_v, posy_v, posz_v, batch_v, cell_v,
             srcb, dstb, sxb, syb, szb, d2b, zb, embb, sem):
        c = lax.axis_index("c")
        s = lax.axis_index("s")
        wid = s * NC + c
        pltpu.sync_copy(posT_h.at[0], posx_v)
        pltpu.sync_copy(posT_h.at[1], posy_v)
        pltpu.sync_copy(posT_h.at[2], posz_v)
        pltpu.sync_copy(batch_h, batch_v)
        pltpu.sync_copy(cellf_h, cell_v)
        base = wid * EPT

        def echunk(ch, carry):
            off = base + ch * CK
            pltpu.sync_copy(src_h.at[pl.ds(off, CK)], srcb)
            pltpu.sync_copy(dst_h.at[pl.ds(off, CK)], dstb)
            pltpu.sync_copy(shiftsT_h.at[0, pl.ds(off, CK)], sxb)
            pltpu.sync_copy(shiftsT_h.at[1, pl.ds(off, CK)], syb)
            pltpu.sync_copy(shiftsT_h.at[2, pl.ds(off, CK)], szb)
            for g in range(CK // LN):
                sl = pl.ds(g * LN, LN)
                s16 = srcb[sl]
                t16 = dstb[sl]
                pxs = plsc.load_gather(posx_v, [s16])
                pys = plsc.load_gather(posy_v, [s16])
                pzs = plsc.load_gather(posz_v, [s16])
                pxt = plsc.load_gather(posx_v, [t16])
                pyt = plsc.load_gather(posy_v, [t16])
                pzt = plsc.load_gather(posz_v, [t16])
                b16 = plsc.load_gather(batch_v, [s16]) * 9
                cm = [plsc.load_gather(cell_v, [b16 + k]) for k in range(9)]
                sx = sxb[sl]
                sy = syb[sl]
                sz = szb[sl]
                ex = pxt - pxs + sx * cm[0] + sy * cm[3] + sz * cm[6]
                ey = pyt - pys + sx * cm[1] + sy * cm[4] + sz * cm[7]
                ez = pzt - pzs + sx * cm[2] + sy * cm[5] + sz * cm[8]
                d2 = jnp.maximum(ex * ex + ey * ey + ez * ez, 1e-12)
                d2b[sl] = d2
            pltpu.sync_copy(d2b, d2_h.at[pl.ds(off, CK)])
            return carry

        lax.fori_loop(0, n_ech, echunk, 0)

        for r in range(n_rounds):
            cid = wid + NW * r

            def emb_round(cid=cid):
                off = cid * CK
                pltpu.sync_copy(z_h.at[pl.ds(off, CK)], zb)
                pltpu.async_copy(emb_h.at[zb], embb, sem).wait()
                pltpu.sync_copy(embb, h0_h.at[pl.ds(off, CK)])

            pl.when(cid < n_nch)(emb_round)

    return geom


def _build_filter(E, G, F, BE):
    grid = E // BE
    gstep = CUTOFF / (G - 1)
    coeff = -0.5 / gstep ** 2

    def body(d2_ref, w1_ref, b1_ref, w2_ref, b2_ref, o0_ref, o1_ref, o2_ref):
        d2 = d2_ref[...]
        d = jnp.sqrt(d2)
        offs = lax.broadcasted_iota(jnp.float32, (1, G), 1) * gstep
        ea = jnp.exp(coeff * (d - offs) ** 2)
        s2 = d2 * (1.0 / (CUTOFF * CUTOFF))
        inside = s2 < 1.0
        denom = jnp.where(inside, jnp.maximum(1.0 - s2, 1e-3), 1.0)
        env = jnp.where(inside, jnp.exp(1.0 - 1.0 / denom), 0.0)
        ea = ea * env
        Cf = 0.5 * (jnp.cos(d * (math.pi / CUTOFF)) + 1.0)
        w1 = w1_ref[...]
        b1 = b1_ref[...]
        w2 = w2_ref[...]
        b2 = b2_ref[...]
        outs = (o0_ref, o1_ref, o2_ref)
        for t in range(3):
            W = jnp.dot(ea, w1[t], preferred_element_type=jnp.float32) + b1[t][None, :]
            W = _ssp(W)
            W = jnp.dot(W, w2[t], preferred_element_type=jnp.float32) + b2[t][None, :]
            outs[t][...] = W * Cf

    return pl.pallas_call(
        body,
        grid=(grid,),
        in_specs=[
            pl.BlockSpec((BE, 1), lambda i: (i, 0)),
            pl.BlockSpec((3, G, F), lambda i: (0, 0, 0)),
            pl.BlockSpec((3, F), lambda i: (0, 0)),
            pl.BlockSpec((3, F, F), lambda i: (0, 0, 0)),
            pl.BlockSpec((3, F), lambda i: (0, 0)),
        ],
        out_specs=[pl.BlockSpec((BE, F), lambda i: (i, 0))] * 3,
        out_shape=[jax.ShapeDtypeStruct((E, F), jnp.float32)] * 3,
    )


def _build_mp(N, E, H, CK):
    EPT = E // NW
    n_ch = EPT // CK
    RPS = N // NS
    ZR = 125
    mesh = plsc.VectorSubcoreMesh(core_axis_name="c", subcore_axis_name="s")

    @functools.partial(
        pl.kernel,
        out_type=jax.ShapeDtypeStruct((NC, N, H), jnp.float32),
        mesh=mesh,
        scratch_types=[
            pltpu.VMEM_SHARED((N, H), jnp.float32),
            pltpu.VMEM((CK,), jnp.int32),
            pltpu.VMEM((CK,), jnp.int32),
            pltpu.VMEM((CK, H), jnp.float32),
            pltpu.VMEM((CK, H), jnp.float32),
            pltpu.VMEM((125, H), jnp.float32),
            pltpu.SemaphoreType.DMA,
        ],
    )
    def mp(hh_h, w_h, src_h, dst_h, out_h, acc, srcb, dstb, hhb, wb, zrows, sem):
        c = lax.axis_index("c")
        s = lax.axis_index("s")
        wid = s * NC + c

        def zr(i, carry):
            for k in range(H // LN):
                zrows[i, pl.ds(k * LN, LN)] = jnp.zeros((LN,), jnp.float32)
            return carry

        lax.fori_loop(0, 125, zr, 0)
        for j in range(RPS // 125):
            pltpu.sync_copy(zrows, acc.at[pl.ds(s * RPS + j * 125, 125)])
        plsc.subcore_barrier()

        base = wid * EPT

        def chunk(ch, carry):
            off = base + ch * CK
            pltpu.sync_copy(src_h.at[pl.ds(off, CK)], srcb)
            pltpu.sync_copy(dst_h.at[pl.ds(off, CK)], dstb)
            pltpu.sync_copy(w_h.at[pl.ds(off, CK)], wb)
            pltpu.async_copy(hh_h.at[srcb], hhb, sem).wait()

            def mul(e, carry2):
                for k in range(H // LN):
                    sl = pl.ds(k * LN, LN)
                    hhb[e, sl] = hhb[e, sl] * wb[e, sl]
                return carry2

            lax.fori_loop(0, CK, mul, 0)
            pltpu.sync_copy(hhb, acc.at[dstb], add=True)
            return carry

        lax.fori_loop(0, n_ch, chunk, 0)
        plsc.subcore_barrier()
        for j in range(RPS // 125):
            r0 = s * RPS + j * 125
            pltpu.sync_copy(acc.at[pl.ds(r0, 125)], out_h.at[c, pl.ds(r0, 125)])

    return mp


def _build_matmul(N, H, BN):
    def body(h_ref, w_ref, o_ref):
        o_ref[...] = jnp.dot(h_ref[...], w_ref[...],
                             preferred_element_type=jnp.float32)

    return pl.pallas_call(
        body,
        grid=(N // BN,),
        in_specs=[
            pl.BlockSpec((BN, H), lambda i: (i, 0)),
            pl.BlockSpec((H, H), lambda i: (0, 0)),
        ],
        out_specs=pl.BlockSpec((BN, H), lambda i: (i, 0)),
        out_shape=jax.ShapeDtypeStruct((N, H), jnp.float32),
    )


def _build_node(N, H, BN):
    def body(agg_ref, h_ref, w2_ref, b2_ref, iw_ref, ib_ref, nw_ref,
             hn_ref, hh_ref):
        agg = agg_ref[0] + agg_ref[1]
        x = jnp.dot(agg, w2_ref[...], preferred_element_type=jnp.float32)
        x = _ssp(x + b2_ref[...])
        x = jnp.dot(x, iw_ref[...], preferred_element_type=jnp.float32)
        hn = h_ref[...] + x + ib_ref[...]
        hn_ref[...] = hn
        hh_ref[...] = jnp.dot(hn, nw_ref[...],
                              preferred_element_type=jnp.float32)

    return pl.pallas_call(
        body,
        grid=(N // BN,),
        in_specs=[
            pl.BlockSpec((NC, BN, H), lambda i: (0, i, 0)),
            pl.BlockSpec((BN, H), lambda i: (i, 0)),
            pl.BlockSpec((H, H), lambda i: (0, 0)),
            pl.BlockSpec((1, H), lambda i: (0, 0)),
            pl.BlockSpec((H, H), lambda i: (0, 0)),
            pl.BlockSpec((1, H), lambda i: (0, 0)),
            pl.BlockSpec((H, H), lambda i: (0, 0)),
        ],
        out_specs=[pl.BlockSpec((BN, H), lambda i: (i, 0))] * 2,
        out_shape=[jax.ShapeDtypeStruct((N, H), jnp.float32)] * 2,
    )


def _build_readout(N, H, NB, BN):
    def body(h_ref, bt_ref, w1_ref, b1_ref, w2_ref, b2_ref, o_ref):
        h = h_ref[...]
        y = _ssp(jnp.dot(h, w1_ref[...], preferred_element_type=jnp.float32)
                 + b1_ref[...])
        y = jnp.dot(y, w2_ref[...], preferred_element_type=jnp.float32) \
            + b2_ref[...]
        bt = bt_ref[...]
        oh = (bt == lax.broadcasted_iota(jnp.int32, (1, NB), 1))
        part = jnp.sum(y * oh.astype(jnp.float32), axis=0, keepdims=True)

        @pl.when(pl.program_id(0) == 0)
        def _():
            o_ref[...] = jnp.zeros_like(o_ref)

        o_ref[...] += part

    return pl.pallas_call(
        body,
        grid=(N // BN,),
        in_specs=[
            pl.BlockSpec((BN, H), lambda i: (i, 0)),
            pl.BlockSpec((BN, 1), lambda i: (i, 0)),
            pl.BlockSpec((H, H // 2), lambda i: (0, 0)),
            pl.BlockSpec((1, H // 2), lambda i: (0, 0)),
            pl.BlockSpec((H // 2, 1), lambda i: (0, 0)),
            pl.BlockSpec((1, 1), lambda i: (0, 0)),
        ],
        out_specs=pl.BlockSpec((1, NB), lambda i: (0, 0)),
        out_shape=jax.ShapeDtypeStruct((1, NB), jnp.float32),
    )


def kernel(z, pos, edge_index, shifts, cell, batch, emb,
           mlp_w1, mlp_b1, mlp_w2, mlp_b2,
           cf_w1, cf_w2, cf_b2, int_w, int_b,
           lin1_w, lin1_b, lin2_w, lin2_b):
    N = pos.shape[0]
    E = edge_index.shape[1]
    H = emb.shape[1]
    NB = cell.shape[0]
    G = mlp_w1.shape[1]
    F = mlp_w1.shape[2]
    CK = 80

    src = edge_index[0].astype(jnp.int32)
    dst = edge_index[1].astype(jnp.int32)
    posT = pos.T.astype(jnp.float32)
    shiftsT = shifts.T.astype(jnp.float32)
    cellf = cell.reshape(-1).astype(jnp.float32)
    batch32 = batch.astype(jnp.int32)
    z32 = z.astype(jnp.int32)

    geom = _build_geom(N, E, H, NB, CK)
    d2, h0 = geom(posT, batch32, cellf, src, dst, shiftsT, z32, emb)

    filt = _build_filter(E, G, F, BE=512)
    Ws = filt(d2.reshape(E, 1), mlp_w1, mlp_b1, mlp_w2, mlp_b2)

    mp = _build_mp(N, E, H, CK)
    node = _build_node(N, H, BN=1000)
    hh = _build_matmul(N, H, BN=1000)(h0, cf_w1[0])
    h = h0
    for t in range(3):
        parts = mp(hh, Ws[t], src, dst)
        h, hh = node(parts, h, cf_w2[t], cf_b2[t].reshape(1, H),
                     int_w[t], int_b[t].reshape(1, H), cf_w1[(t + 1) % 3])

    ro = _build_readout(N, H, NB, BN=1000)
    energy = ro(h, batch32.reshape(N, 1), lin1_w, lin1_b.reshape(1, -1),
                lin2_w, lin2_b.reshape(1, 1))
    return energy.reshape(NB, 1)


# speedup vs baseline: 2.8335x; 2.8335x over previous
import functools
import math

import jax
import jax.numpy as jnp
from jax import lax
from jax.experimental import pallas as pl
from jax.experimental.pallas import tpu as pltpu
from jax.experimental.pallas import tpu_sc as plsc

NC = 2
NS = 16
NW = NC * NS
LN = 16

CUTOFF = 5.0
LOG2 = math.log(2.0)


def _ssp(x):
    return jnp.maximum(x, 0.0) + jnp.log1p(jnp.exp(-jnp.abs(x))) - LOG2


def _build_geom(N, E, H, NB, CK):
    EPT = E // NW
    n_ech = EPT // CK
    n_nch = N // CK
    n_rounds = (n_nch + NW - 1) // NW
    mesh = plsc.VectorSubcoreMesh(core_axis_name="c", subcore_axis_name="s")

    @functools.partial(
        pl.kernel,
        out_type=(jax.ShapeDtypeStruct((E,), jnp.float32),
                  jax.ShapeDtypeStruct((N, H), jnp.float32)),
        mesh=mesh,
        compiler_params=pltpu.CompilerParams(needs_layout_passes=False),
        scratch_types=[
            pltpu.VMEM((N,), jnp.float32),
            pltpu.VMEM((N,), jnp.float32),
            pltpu.VMEM((N,), jnp.float32),
            pltpu.VMEM((N,), jnp.int32),
            pltpu.VMEM((NB * 9,), jnp.float32),
            pltpu.VMEM((CK,), jnp.int32),
            pltpu.VMEM((CK,), jnp.int32),
            pltpu.VMEM((CK,), jnp.float32),
            pltpu.VMEM((CK,), jnp.float32),
            pltpu.VMEM((CK,), jnp.float32),
            pltpu.VMEM((CK,), jnp.float32),
            pltpu.VMEM((CK,), jnp.int32),
            pltpu.VMEM((CK, H), jnp.float32),
            pltpu.SemaphoreType.DMA,
        ],
    )
    def geom(posx_h, posy_h, posz_h, batch_h, cellf_h, src_h, dst_h,
             sx_h, sy_h, sz_h, z_h, emb_h,
             d2_h, h0_h,
             posx_v, posy_v, posz_v, batch_v, cell_v,
             srcb, dstb, sxb, syb, szb, d2b, zb, embb, sem):
        c = lax.axis_index("c")
        s = lax.axis_index("s")
        wid = s * NC + c
        pltpu.sync_copy(posx_h, posx_v)
        pltpu.sync_copy(posy_h, posy_v)
        pltpu.sync_copy(posz_h, posz_v)
        pltpu.sync_copy(batch_h, batch_v)
        pltpu.sync_copy(cellf_h, cell_v)
        base = wid * EPT

        def echunk(ch, carry):
            off = base + ch * CK
            pltpu.sync_copy(src_h.at[pl.ds(off, CK)], srcb)
            pltpu.sync_copy(dst_h.at[pl.ds(off, CK)], dstb)
            pltpu.sync_copy(sx_h.at[pl.ds(off, CK)], sxb)
            pltpu.sync_copy(sy_h.at[pl.ds(off, CK)], syb)
            pltpu.sync_copy(sz_h.at[pl.ds(off, CK)], szb)
            for g in range(CK // LN):
                sl = pl.ds(g * LN, LN)
                s16 = srcb[sl]
                t16 = dstb[sl]
                pxs = plsc.load_gather(posx_v, [s16])
                pys = plsc.load_gather(posy_v, [s16])
                pzs = plsc.load_gather(posz_v, [s16])
                pxt = plsc.load_gather(posx_v, [t16])
                pyt = plsc.load_gather(posy_v, [t16])
                pzt = plsc.load_gather(posz_v, [t16])
                b16 = plsc.load_gather(batch_v, [s16]) * 9
                cm = [plsc.load_gather(cell_v, [b16 + k]) for k in range(9)]
                sx = sxb[sl]
                sy = syb[sl]
                sz = szb[sl]
                ex = pxt - pxs + sx * cm[0] + sy * cm[3] + sz * cm[6]
                ey = pyt - pys + sx * cm[1] + sy * cm[4] + sz * cm[7]
                ez = pzt - pzs + sx * cm[2] + sy * cm[5] + sz * cm[8]
                d2 = jnp.maximum(ex * ex + ey * ey + ez * ez, 1e-12)
                d2b[sl] = d2
            pltpu.sync_copy(d2b, d2_h.at[pl.ds(off, CK)])
            return carry

        lax.fori_loop(0, n_ech, echunk, 0)

        for r in range(n_rounds):
            cid = wid + NW * r

            def emb_round(cid=cid):
                off = cid * CK
                pltpu.sync_copy(z_h.at[pl.ds(off, CK)], zb)
                pltpu.async_copy(emb_h.at[zb], embb, sem).wait()
                pltpu.sync_copy(embb, h0_h.at[pl.ds(off, CK)])

            pl.when(cid < n_nch)(emb_round)

    return geom


def _build_filter(E, G, F, BE):
    grid = E // BE
    gstep = CUTOFF / (G - 1)
    coeff = -0.5 / gstep ** 2

    def body(d2_ref, w1_ref, b1_ref, w2_ref, b2_ref, o0_ref, o1_ref, o2_ref):
        d2 = d2_ref[...]
        d = jnp.sqrt(d2)
        offs = lax.broadcasted_iota(jnp.int32, (1, G), 1).astype(jnp.float32) * gstep
        ea = jnp.exp(coeff * (d - offs) ** 2)
        s2 = d2 * (1.0 / (CUTOFF * CUTOFF))
        inside = s2 < 1.0
        denom = jnp.where(inside, jnp.maximum(1.0 - s2, 1e-3), 1.0)
        env = jnp.where(inside, jnp.exp(1.0 - 1.0 / denom), 0.0)
        ea = ea * env
        Cf = 0.5 * (jnp.cos(d * (math.pi / CUTOFF)) + 1.0)
        w1 = w1_ref[...]
        b1 = b1_ref[...]
        w2 = w2_ref[...]
        b2 = b2_ref[...]
        outs = (o0_ref, o1_ref, o2_ref)
        for t in range(3):
            W = jnp.dot(ea, w1[t], preferred_element_type=jnp.float32) + b1[t][None, :]
            W = _ssp(W)
            W = jnp.dot(W, w2[t], preferred_element_type=jnp.float32) + b2[t][None, :]
            outs[t][...] = W * Cf

    return pl.pallas_call(
        body,
        grid=(grid,),
        in_specs=[
            pl.BlockSpec((BE, 1), lambda i: (i, 0)),
            pl.BlockSpec((3, G, F), lambda i: (0, 0, 0)),
            pl.BlockSpec((3, F), lambda i: (0, 0)),
            pl.BlockSpec((3, F, F), lambda i: (0, 0, 0)),
            pl.BlockSpec((3, F), lambda i: (0, 0)),
        ],
        out_specs=[pl.BlockSpec((BE, F), lambda i: (i, 0))] * 3,
        out_shape=[jax.ShapeDtypeStruct((E, F), jnp.float32)] * 3,
    )


def _build_mp(N, E, H, CK):
    EPT = E // NW
    n_ch = EPT // CK
    CHR = 200
    n_rch = N // CHR
    n_rrounds = (n_rch + NS - 1) // NS
    mesh = plsc.VectorSubcoreMesh(core_axis_name="c", subcore_axis_name="s")

    @functools.partial(
        pl.kernel,
        out_type=jax.ShapeDtypeStruct((NC, N, H), jnp.float32),
        mesh=mesh,
        compiler_params=pltpu.CompilerParams(needs_layout_passes=False),
        scratch_types=[
            pltpu.VMEM_SHARED((N, H), jnp.float32),
            pltpu.VMEM((CK,), jnp.int32),
            pltpu.VMEM((CK,), jnp.int32),
            pltpu.VMEM((CK, H), jnp.float32),
            pltpu.VMEM((CK, H), jnp.float32),
            pltpu.VMEM((CHR, H), jnp.float32),
            pltpu.SemaphoreType.DMA,
        ],
    )
    def mp(hh_h, w_h, src_h, dst_h, out_h, acc, srcb, dstb, hhb, wb, zrows, sem):
        c = lax.axis_index("c")
        s = lax.axis_index("s")
        wid = s * NC + c

        def zr(i, carry):
            for k in range(H // LN):
                zrows[i, pl.ds(k * LN, LN)] = jnp.zeros((LN,), jnp.float32)
            return carry

        lax.fori_loop(0, CHR, zr, 0)
        for r in range(n_rrounds):
            cid = s + NS * r

            def zero_round(cid=cid):
                pltpu.sync_copy(zrows, acc.at[pl.ds(cid * CHR, CHR)])

            pl.when(cid < n_rch)(zero_round)
        plsc.subcore_barrier()

        base = wid * EPT

        def chunk(ch, carry):
            off = base + ch * CK
            pltpu.sync_copy(src_h.at[pl.ds(off, CK)], srcb)
            pltpu.sync_copy(dst_h.at[pl.ds(off, CK)], dstb)
            pltpu.sync_copy(w_h.at[pl.ds(off, CK)], wb)
            pltpu.async_copy(hh_h.at[srcb], hhb, sem).wait()

            def mul(e, carry2):
                for k in range(H // LN):
                    sl = pl.ds(k * LN, LN)
                    hhb[e, sl] = hhb[e, sl] * wb[e, sl]
                return carry2

            lax.fori_loop(0, CK, mul, 0)
            pltpu.sync_copy(hhb, acc.at[dstb], add=True)
            return carry

        lax.fori_loop(0, n_ch, chunk, 0)
        plsc.subcore_barrier()
        for r in range(n_rrounds):
            cid = s + NS * r

            def out_round(cid=cid):
                r0 = cid * CHR
                pltpu.sync_copy(acc.at[pl.ds(r0, CHR)], zrows)
                pltpu.sync_copy(zrows, out_h.at[c, pl.ds(r0, CHR)])

            pl.when(cid < n_rch)(out_round)

    return mp


def _build_matmul(N, H, BN):
    def body(h_ref, w_ref, o_ref):
        o_ref[...] = jnp.dot(h_ref[...], w_ref[...],
                             preferred_element_type=jnp.float32)

    return pl.pallas_call(
        body,
        grid=(N // BN,),
        in_specs=[
            pl.BlockSpec((BN, H), lambda i: (i, 0)),
            pl.BlockSpec((H, H), lambda i: (0, 0)),
        ],
        out_specs=pl.BlockSpec((BN, H), lambda i: (i, 0)),
        out_shape=jax.ShapeDtypeStruct((N, H), jnp.float32),
    )


def _build_node(N, H, BN):
    def body(agg_ref, h_ref, w2_ref, b2_ref, iw_ref, ib_ref, nw_ref,
             hn_ref, hh_ref):
        agg = agg_ref[0] + agg_ref[1]
        x = jnp.dot(agg, w2_ref[...], preferred_element_type=jnp.float32)
        x = _ssp(x + b2_ref[...])
        x = jnp.dot(x, iw_ref[...], preferred_element_type=jnp.float32)
        hn = h_ref[...] + x + ib_ref[...]
        hn_ref[...] = hn
        hh_ref[...] = jnp.dot(hn, nw_ref[...],
                              preferred_element_type=jnp.float32)

    return pl.pallas_call(
        body,
        grid=(N // BN,),
        in_specs=[
            pl.BlockSpec((NC, BN, H), lambda i: (0, i, 0)),
            pl.BlockSpec((BN, H), lambda i: (i, 0)),
            pl.BlockSpec((H, H), lambda i: (0, 0)),
            pl.BlockSpec((1, H), lambda i: (0, 0)),
            pl.BlockSpec((H, H), lambda i: (0, 0)),
            pl.BlockSpec((1, H), lambda i: (0, 0)),
            pl.BlockSpec((H, H), lambda i: (0, 0)),
        ],
        out_specs=[pl.BlockSpec((BN, H), lambda i: (i, 0))] * 2,
        out_shape=[jax.ShapeDtypeStruct((N, H), jnp.float32)] * 2,
    )


def _build_readout(N, H, NB, BN):
    def body(h_ref, bt_ref, w1_ref, b1_ref, w2_ref, b2_ref, o_ref):
        h = h_ref[...]
        y = _ssp(jnp.dot(h, w1_ref[...], preferred_element_type=jnp.float32)
                 + b1_ref[...])
        y = jnp.dot(y, w2_ref[...], preferred_element_type=jnp.float32) \
            + b2_ref[...]
        bt = bt_ref[...]
        oh = (bt == lax.broadcasted_iota(jnp.int32, (1, NB), 1))
        part = jnp.sum(y * oh.astype(jnp.float32), axis=0, keepdims=True)

        @pl.when(pl.program_id(0) == 0)
        def _():
            o_ref[...] = jnp.zeros_like(o_ref)

        o_ref[...] += part

    return pl.pallas_call(
        body,
        grid=(N // BN,),
        in_specs=[
            pl.BlockSpec((BN, H), lambda i: (i, 0)),
            pl.BlockSpec((BN, 1), lambda i: (i, 0)),
            pl.BlockSpec((H, H // 2), lambda i: (0, 0)),
            pl.BlockSpec((1, H // 2), lambda i: (0, 0)),
            pl.BlockSpec((H // 2, 1), lambda i: (0, 0)),
            pl.BlockSpec((1, 1), lambda i: (0, 0)),
        ],
        out_specs=pl.BlockSpec((1, NB), lambda i: (0, 0)),
        out_shape=jax.ShapeDtypeStruct((1, NB), jnp.float32),
    )


def kernel(z, pos, edge_index, shifts, cell, batch, emb,
           mlp_w1, mlp_b1, mlp_w2, mlp_b2,
           cf_w1, cf_w2, cf_b2, int_w, int_b,
           lin1_w, lin1_b, lin2_w, lin2_b):
    N = pos.shape[0]
    E = edge_index.shape[1]
    H = emb.shape[1]
    NB = cell.shape[0]
    G = mlp_w1.shape[1]
    F = mlp_w1.shape[2]
    CK = 80

    src = edge_index[0].astype(jnp.int32)
    dst = edge_index[1].astype(jnp.int32)
    posx = pos[:, 0].astype(jnp.float32)
    posy = pos[:, 1].astype(jnp.float32)
    posz = pos[:, 2].astype(jnp.float32)
    sx = shifts[:, 0].astype(jnp.float32)
    sy = shifts[:, 1].astype(jnp.float32)
    sz = shifts[:, 2].astype(jnp.float32)
    cellf = cell.reshape(-1).astype(jnp.float32)
    batch32 = batch.astype(jnp.int32)
    z32 = z.astype(jnp.int32)

    geom = _build_geom(N, E, H, NB, CK)
    d2, h0 = geom(posx, posy, posz, batch32, cellf, src, dst,
                  sx, sy, sz, z32, emb)

    filt = _build_filter(E, G, F, BE=512)
    Ws = filt(d2.reshape(E, 1), mlp_w1, mlp_b1, mlp_w2, mlp_b2)

    mp = _build_mp(N, E, H, CK)
    node = _build_node(N, H, BN=1000)
    hh = _build_matmul(N, H, BN=1000)(h0, cf_w1[0])
    h = h0
    for t in range(3):
        parts = mp(hh, Ws[t], src, dst)
        h, hh = node(parts, h, cf_w2[t], cf_b2[t].reshape(1, H),
                     int_w[t], int_b[t].reshape(1, H), cf_w1[(t + 1) % 3])

    ro = _build_readout(N, H, NB, BN=1000)
    energy = ro(h, batch32.reshape(N, 1), lin1_w, lin1_b.reshape(1, -1),
                lin2_w, lin2_b.reshape(1, 1))
    return energy.reshape(NB, 1)
